# baseline (speedup 1.0000x reference)
import functools

import jax
import jax.numpy as jnp
from jax import lax
from jax.experimental import pallas as pl
from jax.experimental.pallas import tpu as pltpu

K_WINNERS = 32


def _fused_body(x_ref, wp_ref, w2_ref, b2_ref, out_ref, y_scr):
    y = lax.dot_general(
        x_ref[...], wp_ref[...],
        (((1,), (1,)), ((), ())),
        preferred_element_type=jnp.float32,
    )
    y_scr[...] = y

    rowmax = jnp.max(y, axis=1, keepdims=True)
    hi_row = lax.bitcast_convert_type(rowmax, jnp.int32) + 1

    SUB = 1280
    ysub = y_scr[:, :SUB]

    def body_a(_, carry):
        lo, hi = carry
        mid = lo + (hi - lo) // 2
        midf = lax.bitcast_convert_type(mid, jnp.float32)
        cnt = jnp.sum(
            (ysub >= midf).astype(jnp.float32), axis=1, keepdims=True
        )
        pred = cnt >= K_WINNERS
        return jnp.where(pred, mid, lo), jnp.where(pred, hi, mid)

    lo_a, _ = lax.fori_loop(
        0, 31, body_a, (jnp.zeros_like(hi_row), hi_row)
    )

    ones_n = jnp.ones((1, y.shape[1]), jnp.float32)
    kf = jnp.float32(K_WINNERS)

    def count_ge(midf):
        maskf = (y_scr[...] >= midf).astype(jnp.float32)
        return lax.dot_general(
            maskf, ones_n,
            (((1,), (1,)), ((), ())),
            preferred_element_type=jnp.float32,
        )

    cnt_lo0 = count_ge(lax.bitcast_convert_type(lo_a, jnp.float32))

    def active_rows(lo, hi, cl):
        return jnp.logical_and(cl != kf, hi - lo > 1)

    def cond_b(carry):
        lo, hi, cl, ch, it = carry
        return jnp.any(active_rows(lo, hi, cl))

    def body_b(carry):
        lo, hi, cl, ch, it = carry
        act = active_rows(lo, hi, cl)
        lo_f = lax.bitcast_convert_type(lo, jnp.float32)
        hi_f = lax.bitcast_convert_type(hi, jnp.float32)
        frac = (cl - kf) / jnp.maximum(cl - ch, 1.0)
        mid_interp = lax.bitcast_convert_type(
            lo_f + (hi_f - lo_f) * frac, jnp.int32
        )
        mid_bisect = lo + (hi - lo) // 2
        mid = jnp.where(it % 2 == 0, mid_interp, mid_bisect)
        mid = jnp.clip(mid, lo + 1, hi - 1)
        cnt = count_ge(lax.bitcast_convert_type(mid, jnp.float32))
        pred = cnt >= kf
        lo2 = jnp.where(pred, mid, lo)
        hi2 = jnp.where(pred, hi, mid)
        cl2 = jnp.where(pred, cnt, cl)
        ch2 = jnp.where(pred, ch, cnt)
        return (
            jnp.where(act, lo2, lo), jnp.where(act, hi2, hi),
            jnp.where(act, cl2, cl), jnp.where(act, ch2, ch),
            it + 1,
        )

    lo, hi, cl, ch, _ = lax.while_loop(
        cond_b, body_b,
        (lo_a, hi_row, cnt_lo0, jnp.zeros_like(cnt_lo0), jnp.int32(0)),
    )
    lo_f = lax.bitcast_convert_type(lo, jnp.float32)
    cand_min = jnp.min(
        jnp.where(y_scr[...] >= lo_f, y_scr[...], jnp.float32(3.4e38)),
        axis=1, keepdims=True,
    )
    thresh = jnp.where(cl == kf, cand_min, lo_f)

    yv = y_scr[...]
    sparse = jnp.where(yv >= thresh, yv, 0.0)
    w2c = jnp.maximum(w2_ref[...], 0.0)
    out = lax.dot_general(
        sparse, w2c,
        (((1,), (1,)), ((), ())),
        preferred_element_type=jnp.float32,
    )
    out_ref[...] = out + b2_ref[...]


@functools.partial(jax.jit, static_argnames=("bt",))
def _run(x, W_proj, W2, b2, bt=64):
    B, F = x.shape
    N = W_proj.shape[0]
    C = W2.shape[0]
    grid = (B // bt,)
    return pl.pallas_call(
        _fused_body,
        grid=grid,
        in_specs=[
            pl.BlockSpec((bt, F), lambda i: (i, 0)),
            pl.BlockSpec((N, F), lambda i: (0, 0)),
            pl.BlockSpec((C, N), lambda i: (0, 0)),
            pl.BlockSpec((1, C), lambda i: (0, 0)),
        ],
        out_specs=pl.BlockSpec((bt, C), lambda i: (i, 0)),
        out_shape=jax.ShapeDtypeStruct((B, C), jnp.float32),
        scratch_shapes=[pltpu.VMEM((bt, N), jnp.float32)],
        compiler_params=pltpu.CompilerParams(
            vmem_limit_bytes=63 * 1024 * 1024,
        ),
    )(x, W_proj, W2, b2.reshape(1, C))


def kernel(x, W_proj, W2, b2):
    return _run(x, W_proj, W2, b2)


# speedup vs baseline: 1.1349x; 1.1349x over previous
import functools

import jax
import jax.numpy as jnp
from jax import lax
from jax.experimental import pallas as pl
from jax.experimental.pallas import tpu as pltpu

K_WINNERS = 32


def _fused_body(xh_ref, xl_ref, wp_ref, w2_ref, b2_ref, out_ref, y_scr):
    dims = (((1,), (1,)), ((), ()))
    y = lax.dot_general(
        xh_ref[...], wp_ref[...], dims,
        preferred_element_type=jnp.float32,
    ) + lax.dot_general(
        xl_ref[...], wp_ref[...], dims,
        preferred_element_type=jnp.float32,
    )
    y_scr[...] = y
    del y

    rowmax = jnp.max(y_scr[...], axis=1, keepdims=True)
    hi_row = lax.bitcast_convert_type(rowmax, jnp.int32) + 1

    SUB = 1280
    ysub = y_scr[:, :SUB]

    def body_a(_, carry):
        lo, hi = carry
        mid = lo + (hi - lo) // 2
        midf = lax.bitcast_convert_type(mid, jnp.float32)
        cnt = jnp.sum(
            (ysub >= midf).astype(jnp.float32), axis=1, keepdims=True
        )
        pred = cnt >= K_WINNERS
        return jnp.where(pred, mid, lo), jnp.where(pred, hi, mid)

    lo_a, _ = lax.fori_loop(
        0, 31, body_a, (jnp.zeros_like(hi_row), hi_row)
    )

    ones_n = jnp.ones((1, y_scr.shape[1]), jnp.float32)
    kf = jnp.float32(K_WINNERS)

    def count_ge(midf):
        maskf = (y_scr[...] >= midf).astype(jnp.float32)
        return lax.dot_general(
            maskf, ones_n,
            (((1,), (1,)), ((), ())),
            preferred_element_type=jnp.float32,
        )

    cnt_lo0 = count_ge(lax.bitcast_convert_type(lo_a, jnp.float32))

    def active_rows(lo, hi, cl):
        return jnp.logical_and(cl != kf, hi - lo > 1)

    def cond_b(carry):
        lo, hi, cl, ch, it = carry
        return jnp.any(active_rows(lo, hi, cl))

    def body_b(carry):
        lo, hi, cl, ch, it = carry
        act = active_rows(lo, hi, cl)
        lo_f = lax.bitcast_convert_type(lo, jnp.float32)
        hi_f = lax.bitcast_convert_type(hi, jnp.float32)
        frac = (cl - kf) / jnp.maximum(cl - ch, 1.0)
        mid_interp = lax.bitcast_convert_type(
            lo_f + (hi_f - lo_f) * frac, jnp.int32
        )
        mid_bisect = lo + (hi - lo) // 2
        mid = jnp.where(it % 2 == 0, mid_interp, mid_bisect)
        mid = jnp.clip(mid, lo + 1, hi - 1)
        cnt = count_ge(lax.bitcast_convert_type(mid, jnp.float32))
        pred = cnt >= kf
        lo2 = jnp.where(pred, mid, lo)
        hi2 = jnp.where(pred, hi, mid)
        cl2 = jnp.where(pred, cnt, cl)
        ch2 = jnp.where(pred, ch, cnt)
        return (
            jnp.where(act, lo2, lo), jnp.where(act, hi2, hi),
            jnp.where(act, cl2, cl), jnp.where(act, ch2, ch),
            it + 1,
        )

    lo, hi, cl, ch, _ = lax.while_loop(
        cond_b, body_b,
        (lo_a, hi_row, cnt_lo0, jnp.zeros_like(cnt_lo0), jnp.int32(0)),
    )
    thresh = lax.bitcast_convert_type(lo, jnp.float32)

    yv = y_scr[...]
    y_scr[...] = jnp.where(yv >= thresh, yv, 0.0)
    w2c = jnp.maximum(w2_ref[...], 0.0)
    out = lax.dot_general(
        y_scr[...], w2c,
        (((1,), (1,)), ((), ())),
        preferred_element_type=jnp.float32,
    )
    out_ref[...] = out + b2_ref[...]


@functools.partial(jax.jit, static_argnames=("bt",))
def _run(x, W_proj, W2, b2, bt=128):
    B, F = x.shape
    N = W_proj.shape[0]
    C = W2.shape[0]
    grid = (B // bt,)
    run = pl.pallas_call(
        _fused_body,
        grid=grid,
        in_specs=[
            pl.BlockSpec((bt, F), lambda i: (i, 0)),
            pl.BlockSpec((bt, F), lambda i: (i, 0)),
            pl.BlockSpec((N, F), lambda i: (0, 0)),
            pl.BlockSpec((C, N), lambda i: (0, 0)),
            pl.BlockSpec((1, C), lambda i: (0, 0)),
        ],
        out_specs=pl.BlockSpec((bt, C), lambda i: (i, 0)),
        out_shape=jax.ShapeDtypeStruct((B, C), jnp.float32),
        scratch_shapes=[pltpu.VMEM((bt, N), jnp.float32)],
        compiler_params=pltpu.CompilerParams(
            vmem_limit_bytes=63 * 1024 * 1024,
        ),
    )
    x_hi = x.astype(jnp.bfloat16)
    x_lo = (x - x_hi.astype(jnp.float32)).astype(jnp.bfloat16)
    wp_bf = W_proj.astype(jnp.bfloat16)
    return run(x_hi, x_lo, wp_bf, W2, b2.reshape(1, C))


def kernel(x, W_proj, W2, b2):
    return _run(x, W_proj, W2, b2)


# speedup vs baseline: 1.2986x; 1.1442x over previous
import functools

import jax
import jax.numpy as jnp
from jax import lax
from jax.experimental import pallas as pl
from jax.experimental.pallas import tpu as pltpu

K_WINNERS = 32


def _fused_body(xh_ref, xl_ref, wp_ref, w2_ref, b2_ref, out_ref, y_scr):
    dims = (((1,), (1,)), ((), ()))
    y = lax.dot_general(
        xh_ref[...], wp_ref[...], dims,
        preferred_element_type=jnp.float32,
    ) + lax.dot_general(
        xl_ref[...], wp_ref[...], dims,
        preferred_element_type=jnp.float32,
    )
    y_scr[...] = y
    del y

    rowmax = jnp.max(y_scr[...], axis=1, keepdims=True)
    hi_row = lax.bitcast_convert_type(rowmax, jnp.int32) + 1

    SUB = 1280
    ysub = y_scr[:, :SUB]

    def body_a(_, carry):
        lo, hi = carry
        mid = lo + (hi - lo) // 2
        midf = lax.bitcast_convert_type(mid, jnp.float32)
        cnt = jnp.sum(
            (ysub >= midf).astype(jnp.float32), axis=1, keepdims=True
        )
        pred = cnt >= K_WINNERS
        return jnp.where(pred, mid, lo), jnp.where(pred, hi, mid)

    lo_a, _ = lax.fori_loop(
        0, 31, body_a, (jnp.zeros_like(hi_row), hi_row)
    )

    ones_n = jnp.ones((1, y_scr.shape[1]), jnp.float32)
    kf = jnp.float32(K_WINNERS)

    def count_ge(midf):
        maskf = (y_scr[...] >= midf).astype(jnp.float32)
        return lax.dot_general(
            maskf, ones_n,
            (((1,), (1,)), ((), ())),
            preferred_element_type=jnp.float32,
        )

    def cond_b(carry):
        lo, hi = carry
        return jnp.max(hi - lo) > 1

    def body_b(carry):
        lo, hi = carry
        mid = lo + (hi - lo) // 2
        cnt = count_ge(lax.bitcast_convert_type(mid, jnp.float32))
        pred = cnt >= kf
        return jnp.where(pred, mid, lo), jnp.where(pred, hi, mid)

    lo, _ = lax.while_loop(cond_b, body_b, (lo_a, hi_row))
    thresh = lax.bitcast_convert_type(lo, jnp.float32)

    yv = y_scr[...]
    y_scr[...] = jnp.where(yv >= thresh, yv, 0.0)
    w2c = jnp.maximum(w2_ref[...], 0.0)
    out = lax.dot_general(
        y_scr[...], w2c,
        (((1,), (1,)), ((), ())),
        preferred_element_type=jnp.float32,
    )
    out_ref[...] = out + b2_ref[...]


@functools.partial(jax.jit, static_argnames=("bt",))
def _run(x, W_proj, W2, b2, bt=128):
    B, F = x.shape
    N = W_proj.shape[0]
    C = W2.shape[0]
    grid = (B // bt,)
    run = pl.pallas_call(
        _fused_body,
        grid=grid,
        in_specs=[
            pl.BlockSpec((bt, F), lambda i: (i, 0)),
            pl.BlockSpec((bt, F), lambda i: (i, 0)),
            pl.BlockSpec((N, F), lambda i: (0, 0)),
            pl.BlockSpec((C, N), lambda i: (0, 0)),
            pl.BlockSpec((1, C), lambda i: (0, 0)),
        ],
        out_specs=pl.BlockSpec((bt, C), lambda i: (i, 0)),
        out_shape=jax.ShapeDtypeStruct((B, C), jnp.float32),
        scratch_shapes=[pltpu.VMEM((bt, N), jnp.float32)],
        compiler_params=pltpu.CompilerParams(
            vmem_limit_bytes=63 * 1024 * 1024,
        ),
    )
    x_hi = x.astype(jnp.bfloat16)
    x_lo = (x - x_hi.astype(jnp.float32)).astype(jnp.bfloat16)
    wp_bf = W_proj.astype(jnp.bfloat16)
    return run(x_hi, x_lo, wp_bf, W2, b2.reshape(1, C))


def kernel(x, W_proj, W2, b2):
    return _run(x, W_proj, W2, b2)
